# Initial kernel scaffold; baseline (speedup 1.0000x reference)
import jax
import jax.numpy as jnp
from jax.experimental import pallas as pl


def kernel(data, emb_table, lin_W, att_i, att_j, gnn_bias, bn_gamma, bn_beta, W1, W2, b2):
    raise NotImplementedError("write your pallas kernel here")


# speedup vs baseline: 190.7018x; 190.7018x over previous
import functools

import jax
import jax.numpy as jnp
from jax.experimental import pallas as pl
from jax.experimental.pallas import tpu as pltpu

B, N, F, D, TOPK = 32, 1024, 64, 64, 32
BN_EPS = 1e-5
NEG_INF = float("-inf")


def _topk_mask_kernel(emb_ref, idx_ref, mask_ref, cos_ref):
    w = emb_ref[:]
    g = jax.lax.dot_general(
        w, w, (((1,), (1,)), ((), ())), preferred_element_type=jnp.float32
    )
    sq = jnp.sum(w * w, axis=1, keepdims=True)
    nrm = jnp.sqrt(sq)
    row_i = jax.lax.broadcasted_iota(jnp.int32, (N, N), 0)
    col_i = jax.lax.broadcasted_iota(jnp.int32, (N, N), 1)
    eye = jnp.where(row_i == col_i, 1.0, 0.0)
    nrm_row = jax.lax.dot_general(
        nrm, eye, (((0,), (0,)), ((), ())),
        precision=jax.lax.Precision.HIGHEST,
        preferred_element_type=jnp.float32,
    )
    cos_ref[:] = g / (nrm * nrm_row)
    lane = jax.lax.broadcasted_iota(jnp.int32, (N, N), 1)
    mask = jnp.zeros((N, N), jnp.float32)
    for k in range(TOPK):
        c = cos_ref[:]
        rowmax = jnp.max(c, axis=1, keepdims=True)
        amax = jnp.min(
            jnp.where(c == rowmax, lane, N), axis=1, keepdims=True
        )
        idx_ref[:, k : k + 1] = amax
        hit = lane == amax
        mask = mask + jnp.where(hit, 1.0, 0.0)
        cos_ref[:] = jnp.where(hit, NEG_INF, c)
    mask_ref[:] = mask


def _encoder_kernel(
    data_ref, emb_ref, linw_ref, atti_ref, attj_ref, bias_ref,
    gamma_ref, beta_ref, w1_ref, w2_ref, b2_ref, mask_ref, out_ref
):
    d = data_ref[0]
    xt = jax.lax.dot_general(
        d, linw_ref[:], (((0,), (1,)), ((), ())),
        preferred_element_type=jnp.float32,
    )
    emb = emb_ref[:]
    ai_x = atti_ref[:D, :]
    ai_e = atti_ref[D:, :]
    aj_x = attj_ref[:D, :]
    aj_e = attj_ref[D:, :]
    s_i = (
        jnp.dot(xt, ai_x, preferred_element_type=jnp.float32)
        + jnp.dot(emb, ai_e, preferred_element_type=jnp.float32)
    )
    s_j = jax.lax.dot_general(
        aj_x, xt, (((0,), (1,)), ((), ())), preferred_element_type=jnp.float32
    ) + jax.lax.dot_general(
        aj_e, emb, (((0,), (1,)), ((), ())), preferred_element_type=jnp.float32
    )
    logit = s_i + s_j
    logit = jnp.where(logit >= 0.0, logit, 0.2 * logit)
    logit = jnp.where(mask_ref[:] > 0.0, logit, NEG_INF)
    rowmax = jnp.max(logit, axis=1, keepdims=True)
    ex = jnp.exp(logit - rowmax)
    den = jnp.sum(ex, axis=1, keepdims=True)
    attn = ex / den
    agg = jax.lax.dot_general(
        attn, xt, (((1,), (0,)), ((), ())), preferred_element_type=jnp.float32
    )
    agg = agg + bias_ref[:]
    agg = gamma_ref[:] * (agg * (1.0 / (1.0 + BN_EPS) ** 0.5)) + beta_ref[:]
    gcn = jnp.maximum(agg, 0.0)
    p = gcn * emb
    o1 = jax.lax.dot_general(
        p, w1_ref[:], (((0,), (1,)), ((), ())), preferred_element_type=jnp.float32
    )
    o2 = jax.lax.dot_general(
        o1, w2_ref[:], (((1,), (1,)), ((), ())), preferred_element_type=jnp.float32
    )
    out_ref[0] = jax.nn.sigmoid(o2 + b2_ref[:])


def kernel(data, emb_table, lin_W, att_i, att_j, gnn_bias, bn_gamma, bn_beta, W1, W2, b2):
    topk_idx, mask = pl.pallas_call(
        _topk_mask_kernel,
        out_shape=(
            jax.ShapeDtypeStruct((N, TOPK), jnp.int32),
            jax.ShapeDtypeStruct((N, N), jnp.float32),
        ),
        scratch_shapes=[pltpu.VMEM((N, N), jnp.float32)],
    )(jax.lax.stop_gradient(emb_table))

    full = lambda shape: pl.BlockSpec(shape, lambda b: (0,) * len(shape))
    out = pl.pallas_call(
        _encoder_kernel,
        grid=(B,),
        in_specs=[
            pl.BlockSpec((1, F, N), lambda b: (b, 0, 0)),
            full((N, D)),
            full((D, F)),
            full((2 * D, 1)),
            full((2 * D, 1)),
            full((1, D)),
            full((1, D)),
            full((1, D)),
            full((N, N)),
            full((N, N)),
            full((1, N)),
            full((N, N)),
        ],
        out_specs=pl.BlockSpec((1, D, N), lambda b: (b, 0, 0)),
        out_shape=jax.ShapeDtypeStruct((B, D, N), jnp.float32),
    )(
        data,
        emb_table,
        lin_W,
        att_i.reshape(2 * D, 1),
        att_j.reshape(2 * D, 1),
        gnn_bias.reshape(1, D),
        bn_gamma.reshape(1, D),
        bn_beta.reshape(1, D),
        W1,
        W2,
        b2.reshape(1, N),
        mask,
    )
    return (out, emb_table, topk_idx)


# speedup vs baseline: 270.4609x; 1.4182x over previous
import jax
import jax.numpy as jnp
from jax.experimental import pallas as pl
from jax.experimental.pallas import tpu as pltpu

B, N, F, D, TOPK = 32, 1024, 64, 64, 32
BN_EPS = 1e-5
NEG_INF = float("-inf")


def _topk_mask_kernel(emb_ref, w1_ref, w2_ref, idx_ref, mask_ref, w21_ref, cos_ref):
    w = emb_ref[:]
    g = jax.lax.dot_general(
        w, w, (((1,), (1,)), ((), ())), preferred_element_type=jnp.float32
    )
    sq = jnp.sum(w * w, axis=1, keepdims=True)
    nrm = jnp.sqrt(sq)
    row_i = jax.lax.broadcasted_iota(jnp.int32, (N, N), 0)
    col_i = jax.lax.broadcasted_iota(jnp.int32, (N, N), 1)
    eye = jnp.where(row_i == col_i, 1.0, 0.0)
    nrm_row = jax.lax.dot_general(
        nrm, eye, (((0,), (0,)), ((), ())),
        precision=jax.lax.Precision.HIGHEST,
        preferred_element_type=jnp.float32,
    )
    cos_ref[:] = g / (nrm * nrm_row)

    lane = jax.lax.broadcasted_iota(jnp.int32, (N, N), 1)
    for k in range(TOPK):
        c = cos_ref[:]
        amax = jnp.argmax(c, axis=1, keepdims=True).astype(jnp.int32)
        idx_ref[:, k : k + 1] = amax
        cos_ref[:] = jnp.where(lane == amax, NEG_INF, c)
    mask_ref[:] = jnp.where(cos_ref[:] == NEG_INF, 0.0, NEG_INF)

    w21 = jax.lax.dot_general(
        w2_ref[:].astype(jnp.bfloat16), w1_ref[:].astype(jnp.bfloat16),
        (((1,), (0,)), ((), ())), preferred_element_type=jnp.float32,
    )
    w21_ref[:] = w21.astype(jnp.bfloat16)


def _encoder_kernel(
    data_ref, emb_ref, linw_ref, atti_ref, attj_ref, bias_ref,
    gamma_ref, beta_ref, w21_ref, b2_ref, mask_ref, out_ref
):
    d = data_ref[0]
    xt = jax.lax.dot_general(
        d, linw_ref[:], (((0,), (1,)), ((), ())),
        preferred_element_type=jnp.float32,
    )
    emb = emb_ref[:]
    ai_x = atti_ref[:D, :]
    ai_e = atti_ref[D:, :]
    aj_x = attj_ref[:D, :]
    aj_e = attj_ref[D:, :]
    s_i = (
        jnp.dot(xt, ai_x, preferred_element_type=jnp.float32)
        + jnp.dot(emb, ai_e, preferred_element_type=jnp.float32)
    )
    s_j = jax.lax.dot_general(
        aj_x, xt, (((0,), (1,)), ((), ())), preferred_element_type=jnp.float32
    ) + jax.lax.dot_general(
        aj_e, emb, (((0,), (1,)), ((), ())), preferred_element_type=jnp.float32
    )
    sj_max = jnp.max(s_j, axis=1, keepdims=True)
    shift = s_i + sj_max
    shift = jnp.where(shift >= 0.0, shift, 0.2 * shift)
    logit = s_i + s_j
    logit = jnp.where(logit >= 0.0, logit, 0.2 * logit)
    e_bf = jnp.exp(logit - shift + mask_ref[:]).astype(jnp.bfloat16)
    ones_col = jnp.ones((N, 1), jnp.float32)
    xt_ext = jnp.concatenate([xt, ones_col], axis=1).astype(jnp.bfloat16)
    y = jax.lax.dot_general(
        e_bf, xt_ext, (((1,), (0,)), ((), ())), preferred_element_type=jnp.float32
    )
    agg = y[:, :D] * (1.0 / y[:, D:])
    agg = agg + bias_ref[:]
    agg = gamma_ref[:] * (agg * (1.0 / (1.0 + BN_EPS) ** 0.5)) + beta_ref[:]
    gcn = jnp.maximum(agg, 0.0)
    p = (gcn * emb).astype(jnp.bfloat16)
    o = jax.lax.dot_general(
        p, w21_ref[:], (((0,), (1,)), ((), ())), preferred_element_type=jnp.float32
    )
    out_ref[0] = jax.nn.sigmoid(o + b2_ref[:])


def kernel(data, emb_table, lin_W, att_i, att_j, gnn_bias, bn_gamma, bn_beta, W1, W2, b2):
    topk_idx, mask, w21 = pl.pallas_call(
        _topk_mask_kernel,
        out_shape=(
            jax.ShapeDtypeStruct((N, TOPK), jnp.int32),
            jax.ShapeDtypeStruct((N, N), jnp.float32),
            jax.ShapeDtypeStruct((N, N), jnp.bfloat16),
        ),
        scratch_shapes=[pltpu.VMEM((N, N), jnp.float32)],
    )(jax.lax.stop_gradient(emb_table), W1, W2)

    full = lambda shape: pl.BlockSpec(shape, lambda b: (0,) * len(shape))
    out = pl.pallas_call(
        _encoder_kernel,
        grid=(B,),
        in_specs=[
            pl.BlockSpec((1, F, N), lambda b: (b, 0, 0)),
            full((N, D)),
            full((D, F)),
            full((2 * D, 1)),
            full((2 * D, 1)),
            full((1, D)),
            full((1, D)),
            full((1, D)),
            full((N, N)),
            full((1, N)),
            full((N, N)),
        ],
        out_specs=pl.BlockSpec((1, D, N), lambda b: (b, 0, 0)),
        out_shape=jax.ShapeDtypeStruct((B, D, N), jnp.float32),
    )(
        data,
        emb_table,
        lin_W,
        att_i.reshape(2 * D, 1),
        att_j.reshape(2 * D, 1),
        gnn_bias.reshape(1, D),
        bn_gamma.reshape(1, D),
        bn_beta.reshape(1, D),
        w21,
        b2.reshape(1, N),
        mask,
    )
    return (out, emb_table, topk_idx)
